# baseline (speedup 1.0000x reference)
import jax
import jax.numpy as jnp
from jax import lax
from jax.experimental import pallas as pl
from jax.experimental.pallas import tpu as pltpu
from jax.experimental.pallas import tpu_sc as plsc

B, D = 128, 32768
L = 16
NC, NS = 2, 16
NW = NC * NS
ROWS_PER_W = B // NW
CHUNKS = D // L
UNROLL = 8
STEPS = CHUNKS // UNROLL
NBUCKET = 64
BUCKET_SHIFT = 21
FINE = 22
CAP = 8192
FBLK = 128
SHIFT = 20.0


def _body(x_hbm, out_hbm, row_v, cb_v, hist_v):
    c = lax.axis_index("c")
    s = lax.axis_index("s")
    wid = s * NC + c
    zero = jnp.zeros((L,), jnp.float32)
    izero = jnp.zeros((L,), jnp.int32)
    ione = jnp.ones((L,), jnp.int32)

    def do_row(r, _):
        row = wid * ROWS_PER_W + r
        pltpu.sync_copy(x_hbm.at[row], row_v)
        for k in range(NBUCKET // L):
            hist_v[pl.ds(k * L, L)] = zero

        def exp_body(i, accs):
            a0, a1 = accs
            base = i * (UNROLL * L)
            for j in range(UNROLL):
                xv = row_v[pl.ds(base + j * L, L)]
                u = jnp.exp(jnp.minimum(xv - jnp.float32(SHIFT),
                                        jnp.float32(0.0)))
                row_v[pl.ds(base + j * L, L)] = u
                if j % 2 == 0:
                    a0 = a0 + u
                else:
                    a1 = a1 + u
            return a0, a1
        z0, z1 = lax.fori_loop(0, STEPS, exp_body, (zero, zero))
        z_s = jnp.sum(z0 + z1)
        t_thresh = jnp.float32(0.5) * z_s

        lo0 = jnp.maximum(
            lax.bitcast_convert_type(z_s * jnp.float32(2.0 ** -16),
                                     jnp.int32) - jnp.int32(16),
            jnp.int32(0))
        hi0 = lax.bitcast_convert_type(z_s, jnp.int32)

        lo0v = jnp.full((L,), 1, jnp.int32) * lo0
        bmax = jnp.full((L,), NBUCKET - 1, jnp.int32)

        def h_body(i, _unused):
            base = i * (UNROLL * L)
            for j in range(UNROLL):
                v = row_v[pl.ds(base + j * L, L)]
                bits = plsc.bitcast(v, jnp.int32)
                idx = lax.shift_right_arithmetic(bits - lo0v, BUCKET_SHIFT)
                idx = jnp.minimum(jnp.maximum(idx, izero), bmax)
                plsc.addupdate_scatter(hist_v, [idx], v)
            return 0
        lax.fori_loop(0, STEPS, h_body, 0)

        hs = [hist_v[pl.ds(k * L, L)] for k in range(NBUCKET // L)]
        sufs = []
        carry_s = jnp.float32(0.0)
        for k in reversed(range(NBUCKET // L)):
            local = lax.rev(plsc.cumsum(lax.rev(hs[k], (0,))), (0,))
            sufs.insert(0, local + carry_s)
            carry_s = carry_s + jnp.sum(hs[k])
        tv = jnp.full((L,), 1.0, jnp.float32) * t_thresh
        ks = jnp.int32(-1)
        for k in range(NBUCKET // L):
            ks = ks + plsc.all_reduce_population_count(sufs[k] > tv)[0]
        for k in range(NBUCKET // L):
            hist_v[pl.ds(k * L, L)] = sufs[k]
        hist_v[pl.ds(NBUCKET, L)] = zero
        w_hi = plsc.load_gather(
            hist_v, [jnp.minimum(jnp.full((L,), 1, jnp.int32) * (ks + 1),
                                 jnp.full((L,), NBUCKET, jnp.int32))])[0]
        lo = lo0 + ks * jnp.int32(1 << BUCKET_SHIFT) - jnp.int32(1)
        hi = jnp.where(ks == jnp.int32(NBUCKET - 1), hi0,
                       lo + jnp.int32(1 << BUCKET_SHIFT))
        lo_f = lax.bitcast_convert_type(lo, jnp.float32)
        hi_f = lax.bitcast_convert_type(hi, jnp.float32)

        def cp_body(i, off):
            base = i * (UNROLL * L)
            vs, ms, pcs = [], [], []
            for j in range(UNROLL):
                v = row_v[pl.ds(base + j * L, L)]
                msk = (v > lo_f) & (v <= hi_f)
                vs.append(v)
                ms.append(msk)
                pcs.append(plsc.all_reduce_population_count(msk))
            pre = [izero]
            for j in range(1, UNROLL):
                pre.append(pre[j - 1] + pcs[j - 1])
            for j in range(UNROLL):
                cs = plsc.cumsum(jnp.where(ms[j], ione, izero))
                pos = (off + pre[j]) + cs
                pos = jnp.minimum(pos - ione, jnp.full((L,), CAP + FBLK - 1,
                                                       jnp.int32))
                plsc.store_scatter(cb_v, [pos], vs[j], mask=ms[j])
            t01 = (pcs[0] + pcs[1]) + (pcs[2] + pcs[3])
            t23 = (pcs[4] + pcs[5]) + (pcs[6] + pcs[7])
            return off + (t01 + t23)
        off = lax.fori_loop(0, STEPS, cp_body, izero)
        cnt = off[0]
        cpos = jnp.minimum(cnt, jnp.int32(CAP))
        for k in range(FBLK // L):
            plsc.store_scatter(
                cb_v, [cpos + (lax.iota(jnp.int32, L) + jnp.int32(k * L))],
                zero)

        def fine_path(carry):
            lo, hi = carry
            n_blk = lax.shift_right_logical(cnt + jnp.int32(FBLK - 1), 7)

            def fb_body(_, carry):
                lo, hi = carry
                mid = lo + lax.shift_right_logical(hi - lo, 1)
                t = lax.bitcast_convert_type(mid, jnp.float32)

                def w_body(i, accs):
                    a0, a1 = accs
                    base = i * FBLK
                    for j in range(FBLK // L):
                        v = cb_v[pl.ds(base + j * L, L)]
                        w = jnp.where(v > t, v, jnp.float32(0.0))
                        if j % 2 == 0:
                            a0 = a0 + w
                        else:
                            a1 = a1 + w
                    return a0, a1
                w0, w1 = lax.fori_loop(0, n_blk, w_body, (zero, zero))
                W = w_hi + jnp.sum(w0 + w1)
                pred = W > t_thresh
                lo = jnp.where(pred, mid, lo)
                hi = jnp.where(pred, hi, mid)
                return lo, hi
            lo, hi = lax.fori_loop(0, FINE, fb_body, (lo, hi))
            t_lo = lax.bitcast_convert_type(lo, jnp.float32)

            def s_body(i, acc):
                base = i * FBLK
                for j in range(FBLK // L):
                    v = cb_v[pl.ds(base + j * L, L)]
                    acc = acc + jnp.where(v > t_lo, v, jnp.float32(0.0))
                return acc
            sacc = lax.fori_loop(0, n_blk, s_body, zero)
            return lo, hi, w_hi + jnp.sum(sacc)

        def full_path(carry):
            lo, hi = carry

            def fb_body(_, carry):
                lo, hi = carry
                mid = lo + lax.shift_right_logical(hi - lo, 1)
                t = lax.bitcast_convert_type(mid, jnp.float32)

                def w_body(i, accs):
                    a0, a1 = accs
                    base = i * (UNROLL * L)
                    for j in range(UNROLL):
                        v = row_v[pl.ds(base + j * L, L)]
                        w = jnp.where(v > t, v, jnp.float32(0.0))
                        if j % 2 == 0:
                            a0 = a0 + w
                        else:
                            a1 = a1 + w
                    return a0, a1
                w0, w1 = lax.fori_loop(0, STEPS, w_body, (zero, zero))
                W = jnp.sum(w0 + w1)
                pred = W > t_thresh
                lo = jnp.where(pred, mid, lo)
                hi = jnp.where(pred, hi, mid)
                return lo, hi
            lo, hi = lax.fori_loop(0, FINE, fb_body, (lo, hi))

            def s_body(i, acc):
                base = i * (UNROLL * L)
                t_lo = lax.bitcast_convert_type(lo, jnp.float32)
                for j in range(UNROLL):
                    v = row_v[pl.ds(base + j * L, L)]
                    acc = acc + jnp.where(v > t_lo, v, jnp.float32(0.0))
                return acc
            sacc = lax.fori_loop(0, STEPS, s_body, zero)
            return lo, hi, jnp.sum(sacc)

        lo, hi, kept = lax.cond(cnt <= jnp.int32(CAP), fine_path, full_path,
                                (lo, hi))
        t_lo = lax.bitcast_convert_type(lo, jnp.float32)

        kept_v = jnp.full((L,), 1.0, jnp.float32) * kept
        sp_v = kept_v / z_s
        alpha = jnp.full((L,), 1.0, jnp.float32) / (
            z_s * (sp_v + jnp.float32(1e-7)))

        def out_body(i, _unused):
            base = i * (UNROLL * L)
            for j in range(UNROLL):
                v = row_v[pl.ds(base + j * L, L)]
                row_v[pl.ds(base + j * L, L)] = jnp.where(
                    v > t_lo, v * alpha, jnp.float32(0.0))
            return 0
        lax.fori_loop(0, STEPS, out_body, 0)

        pltpu.sync_copy(row_v, out_hbm.at[row])
        return 0

    lax.fori_loop(0, ROWS_PER_W, do_row, 0)


@jax.jit
def kernel(logits):
    return pl.kernel(
        _body,
        out_type=jax.ShapeDtypeStruct((B, D), jnp.float32),
        mesh=plsc.VectorSubcoreMesh(core_axis_name="c", subcore_axis_name="s"),
        scratch_types=[pltpu.VMEM((D,), jnp.float32),
                       pltpu.VMEM((CAP + 2 * FBLK,), jnp.float32),
                       pltpu.VMEM((NBUCKET + L,), jnp.float32)],
        compiler_params=pltpu.CompilerParams(needs_layout_passes=False),
    )(logits)


# speedup vs baseline: 1.6187x; 1.6187x over previous
import jax
import jax.numpy as jnp
from jax import lax
from jax.experimental import pallas as pl
from jax.experimental.pallas import tpu as pltpu
from jax.experimental.pallas import tpu_sc as plsc

B, D = 128, 32768
L = 16
NC, NS = 2, 16
NW = NC * NS
ROWS_PER_W = B // NW
CHUNKS = D // L
UNROLL = 8
STEPS = CHUNKS // UNROLL
COARSE = 6
FINE = 22
CAP = 8192
FBLK = 128
SHIFT = 20.0


def _body(x_hbm, out_hbm, row_v, cb_v):
    c = lax.axis_index("c")
    s = lax.axis_index("s")
    wid = s * NC + c
    zero = jnp.zeros((L,), jnp.float32)
    izero = jnp.zeros((L,), jnp.int32)
    ione = jnp.ones((L,), jnp.int32)

    def wmask(v, t):
        sgn = lax.shift_right_arithmetic(plsc.bitcast(t - v, jnp.int32),
                                         31)
        return plsc.bitcast(plsc.bitcast(v, jnp.int32) & sgn, jnp.float32)

    def do_row(r, _):
        row = wid * ROWS_PER_W + r
        pltpu.sync_copy(x_hbm.at[row], row_v)

        def exp_body(i, accs):
            a0, a1 = accs
            base = i * (UNROLL * L)
            for j in range(UNROLL):
                xv = row_v[pl.ds(base + j * L, L)]
                u = jnp.exp(jnp.minimum(xv - jnp.float32(SHIFT),
                                        jnp.float32(0.0)))
                row_v[pl.ds(base + j * L, L)] = u
                if j % 2 == 0:
                    a0 = a0 + u
                else:
                    a1 = a1 + u
            return a0, a1
        z0, z1 = lax.fori_loop(0, STEPS, exp_body, (zero, zero))
        z_s = jnp.sum(z0 + z1)
        t_thresh = jnp.float32(0.5) * z_s

        lo0 = jnp.maximum(
            lax.bitcast_convert_type(z_s * jnp.float32(2.0 ** -16),
                                     jnp.int32) - jnp.int32(16),
            jnp.int32(0))
        hi0 = lax.bitcast_convert_type(z_s, jnp.int32)

        def bs_body(_, carry):
            lo, hi, above = carry
            mid = lo + lax.shift_right_logical(hi - lo, 1)
            t = lax.bitcast_convert_type(mid, jnp.float32)

            def w_body(i, accs):
                a0, a1, a2, a3 = accs
                base = i * (UNROLL * L)
                for j in range(UNROLL):
                    v = row_v[pl.ds(base + j * L, L)]
                    w = wmask(v, t)
                    if j % 4 == 0:
                        a0 = a0 + w
                    elif j % 4 == 1:
                        a1 = a1 + w
                    elif j % 4 == 2:
                        a2 = a2 + w
                    else:
                        a3 = a3 + w
                return a0, a1, a2, a3
            w0, w1, w2, w3 = lax.fori_loop(0, STEPS, w_body,
                                           (zero, zero, zero, zero))
            W = jnp.sum((w0 + w1) + (w2 + w3))
            pred = W > t_thresh
            lo = jnp.where(pred, mid, lo)
            hi = jnp.where(pred, hi, mid)
            above = jnp.where(pred, above, W)
            return lo, hi, above

        lo, hi, w_hi = lax.fori_loop(0, COARSE, bs_body,
                                     (lo0, hi0, jnp.float32(0.0)))
        lo_f = lax.bitcast_convert_type(lo, jnp.float32)
        hi_f = lax.bitcast_convert_type(hi, jnp.float32)

        def cp_body(i, off):
            base = i * (UNROLL * L)
            vs, ms, pcs = [], [], []
            for j in range(UNROLL):
                v = row_v[pl.ds(base + j * L, L)]
                msk = (v > lo_f) & (v <= hi_f)
                vs.append(v)
                ms.append(msk)
                pcs.append(plsc.all_reduce_population_count(msk))
            pre = [izero]
            for j in range(1, UNROLL):
                pre.append(pre[j - 1] + pcs[j - 1])
            for j in range(UNROLL):
                cs = plsc.cumsum(jnp.where(ms[j], ione, izero))
                pos = (off + pre[j]) + cs
                pos = jnp.minimum(pos - ione, jnp.full((L,), CAP + FBLK - 1,
                                                       jnp.int32))
                plsc.store_scatter(cb_v, [pos], vs[j], mask=ms[j])
            t01 = (pcs[0] + pcs[1]) + (pcs[2] + pcs[3])
            t23 = (pcs[4] + pcs[5]) + (pcs[6] + pcs[7])
            return off + (t01 + t23)
        off = lax.fori_loop(0, STEPS, cp_body, izero)
        cnt = off[0]
        cpos = jnp.minimum(cnt, jnp.int32(CAP))
        for k in range(FBLK // L):
            plsc.store_scatter(
                cb_v, [cpos + (lax.iota(jnp.int32, L) + jnp.int32(k * L))],
                zero)

        def fine_path(carry):
            lo, hi = carry
            n_blk = lax.shift_right_logical(cnt + jnp.int32(FBLK - 1), 7)

            def fb_body(_, carry):
                lo, hi = carry
                mid = lo + lax.shift_right_logical(hi - lo, 1)
                t = lax.bitcast_convert_type(mid, jnp.float32)

                def w_body(i, accs):
                    a0, a1 = accs
                    base = i * FBLK
                    for j in range(FBLK // L):
                        v = cb_v[pl.ds(base + j * L, L)]
                        w = wmask(v, t)
                        if j % 2 == 0:
                            a0 = a0 + w
                        else:
                            a1 = a1 + w
                    return a0, a1
                w0, w1 = lax.fori_loop(0, n_blk, w_body, (zero, zero))
                W = w_hi + jnp.sum(w0 + w1)
                pred = W > t_thresh
                lo = jnp.where(pred, mid, lo)
                hi = jnp.where(pred, hi, mid)
                return lo, hi
            lo, hi = lax.fori_loop(0, FINE, fb_body, (lo, hi))
            t_lo = lax.bitcast_convert_type(lo, jnp.float32)

            def s_body(i, acc):
                base = i * FBLK
                for j in range(FBLK // L):
                    v = cb_v[pl.ds(base + j * L, L)]
                    acc = acc + wmask(v, t_lo)
                return acc
            sacc = lax.fori_loop(0, n_blk, s_body, zero)
            return lo, hi, w_hi + jnp.sum(sacc)

        def full_path(carry):
            lo, hi = carry

            def fb_body(_, carry):
                lo, hi = carry
                mid = lo + lax.shift_right_logical(hi - lo, 1)
                t = lax.bitcast_convert_type(mid, jnp.float32)

                def w_body(i, accs):
                    a0, a1 = accs
                    base = i * (UNROLL * L)
                    for j in range(UNROLL):
                        v = row_v[pl.ds(base + j * L, L)]
                        w = wmask(v, t)
                        if j % 2 == 0:
                            a0 = a0 + w
                        else:
                            a1 = a1 + w
                    return a0, a1
                w0, w1 = lax.fori_loop(0, STEPS, w_body, (zero, zero))
                W = jnp.sum(w0 + w1)
                pred = W > t_thresh
                lo = jnp.where(pred, mid, lo)
                hi = jnp.where(pred, hi, mid)
                return lo, hi
            lo, hi = lax.fori_loop(0, FINE, fb_body, (lo, hi))

            def s_body(i, acc):
                base = i * (UNROLL * L)
                t_lo = lax.bitcast_convert_type(lo, jnp.float32)
                for j in range(UNROLL):
                    v = row_v[pl.ds(base + j * L, L)]
                    acc = acc + wmask(v, t_lo)
                return acc
            sacc = lax.fori_loop(0, STEPS, s_body, zero)
            return lo, hi, jnp.sum(sacc)

        lo, hi, kept = lax.cond(cnt <= jnp.int32(CAP), fine_path, full_path,
                                (lo, hi))
        t_lo = lax.bitcast_convert_type(lo, jnp.float32)

        kept_v = jnp.full((L,), 1.0, jnp.float32) * kept
        sp_v = kept_v / z_s
        alpha = jnp.full((L,), 1.0, jnp.float32) / (
            z_s * (sp_v + jnp.float32(1e-7)))

        def out_body(i, _unused):
            base = i * (UNROLL * L)
            for j in range(UNROLL):
                v = row_v[pl.ds(base + j * L, L)]
                row_v[pl.ds(base + j * L, L)] = jnp.where(
                    v > t_lo, v * alpha, jnp.float32(0.0))
            return 0
        lax.fori_loop(0, STEPS, out_body, 0)

        pltpu.sync_copy(row_v, out_hbm.at[row])
        return 0

    lax.fori_loop(0, ROWS_PER_W, do_row, 0)


@jax.jit
def kernel(logits):
    return pl.kernel(
        _body,
        out_type=jax.ShapeDtypeStruct((B, D), jnp.float32),
        mesh=plsc.VectorSubcoreMesh(core_axis_name="c", subcore_axis_name="s"),
        scratch_types=[pltpu.VMEM((D,), jnp.float32),
                       pltpu.VMEM((CAP + 2 * FBLK,), jnp.float32)],
        compiler_params=pltpu.CompilerParams(needs_layout_passes=False),
    )(logits)


# speedup vs baseline: 1.9545x; 1.2074x over previous
import jax
import jax.numpy as jnp
from jax import lax
from jax.experimental import pallas as pl
from jax.experimental.pallas import tpu as pltpu
from jax.experimental.pallas import tpu_sc as plsc

B, D = 128, 32768
L = 16
NC, NS = 2, 16
NW = NC * NS
ROWS_PER_W = B // NW
CHUNKS = D // L
UNROLL = 8
STEPS = CHUNKS // UNROLL
COARSE = 6
FINE = 22
CAP = 8192
FBLK = 128
SHIFT = 20.0


def _body(x_hbm, out_hbm, b0, b1, b2, cb_v, si0, si1, si2, so0, so1, so2):
    c = lax.axis_index("c")
    s = lax.axis_index("s")
    wid = s * NC + c
    zero = jnp.zeros((L,), jnp.float32)
    izero = jnp.zeros((L,), jnp.int32)
    ione = jnp.ones((L,), jnp.int32)

    def compute_row(row_v):
        def exp_body(i, accs):
            a0, a1 = accs
            base = i * (UNROLL * L)
            for j in range(UNROLL):
                xv = row_v[pl.ds(base + j * L, L)]
                u = jnp.exp(jnp.minimum(xv - jnp.float32(SHIFT),
                                        jnp.float32(0.0)))
                row_v[pl.ds(base + j * L, L)] = u
                if j % 2 == 0:
                    a0 = a0 + u
                else:
                    a1 = a1 + u
            return a0, a1
        z0, z1 = lax.fori_loop(0, STEPS, exp_body, (zero, zero))
        z_s = jnp.sum(z0 + z1)
        t_thresh = jnp.float32(0.5) * z_s

        lo0 = jnp.maximum(
            lax.bitcast_convert_type(z_s * jnp.float32(2.0 ** -16),
                                     jnp.int32) - jnp.int32(16),
            jnp.int32(0))
        hi0 = lax.bitcast_convert_type(z_s, jnp.int32)

        def bs_body(_, carry):
            lo, hi, above = carry
            mid = lo + lax.shift_right_logical(hi - lo, 1)
            t = lax.bitcast_convert_type(mid, jnp.float32)

            def w_body(i, accs):
                a0, a1, a2, a3 = accs
                base = i * (UNROLL * L)
                for j in range(UNROLL):
                    v = row_v[pl.ds(base + j * L, L)]
                    w = jnp.where(v > t, v, jnp.float32(0.0))
                    if j % 4 == 0:
                        a0 = a0 + w
                    elif j % 4 == 1:
                        a1 = a1 + w
                    elif j % 4 == 2:
                        a2 = a2 + w
                    else:
                        a3 = a3 + w
                return a0, a1, a2, a3
            w0, w1, w2, w3 = lax.fori_loop(0, STEPS, w_body,
                                           (zero, zero, zero, zero))
            W = jnp.sum((w0 + w1) + (w2 + w3))
            pred = W > t_thresh
            lo = jnp.where(pred, mid, lo)
            hi = jnp.where(pred, hi, mid)
            above = jnp.where(pred, above, W)
            return lo, hi, above

        lo, hi, w_hi = lax.fori_loop(0, COARSE, bs_body,
                                     (lo0, hi0, jnp.float32(0.0)))
        lo_f = lax.bitcast_convert_type(lo, jnp.float32)
        hi_f = lax.bitcast_convert_type(hi, jnp.float32)

        def cp_body(i, off):
            base = i * (UNROLL * L)
            vs, ms, pcs = [], [], []
            for j in range(UNROLL):
                v = row_v[pl.ds(base + j * L, L)]
                msk = (v > lo_f) & (v <= hi_f)
                vs.append(v)
                ms.append(msk)
                pcs.append(plsc.all_reduce_population_count(msk))
            pre = [izero]
            for j in range(1, UNROLL):
                pre.append(pre[j - 1] + pcs[j - 1])
            for j in range(UNROLL):
                cs = plsc.cumsum(jnp.where(ms[j], ione, izero))
                pos = (off + pre[j]) + cs
                pos = jnp.minimum(pos - ione, jnp.full((L,), CAP + FBLK - 1,
                                                       jnp.int32))
                plsc.store_scatter(cb_v, [pos], vs[j], mask=ms[j])
            t01 = (pcs[0] + pcs[1]) + (pcs[2] + pcs[3])
            t23 = (pcs[4] + pcs[5]) + (pcs[6] + pcs[7])
            return off + (t01 + t23)
        off = lax.fori_loop(0, STEPS, cp_body, izero)
        cnt = off[0]
        cpos = jnp.minimum(cnt, jnp.int32(CAP))
        for k in range(FBLK // L):
            plsc.store_scatter(
                cb_v, [cpos + (lax.iota(jnp.int32, L) + jnp.int32(k * L))],
                zero)

        def fine_path(carry):
            lo, hi = carry
            n_blk = lax.shift_right_logical(cnt + jnp.int32(FBLK - 1), 7)

            def fb_body(_, carry):
                lo, hi = carry
                mid = lo + lax.shift_right_logical(hi - lo, 1)
                t = lax.bitcast_convert_type(mid, jnp.float32)

                def w_body(i, accs):
                    a0, a1 = accs
                    base = i * FBLK
                    for j in range(FBLK // L):
                        v = cb_v[pl.ds(base + j * L, L)]
                        w = jnp.where(v > t, v, jnp.float32(0.0))
                        if j % 2 == 0:
                            a0 = a0 + w
                        else:
                            a1 = a1 + w
                    return a0, a1
                w0, w1 = lax.fori_loop(0, n_blk, w_body, (zero, zero))
                W = w_hi + jnp.sum(w0 + w1)
                pred = W > t_thresh
                lo = jnp.where(pred, mid, lo)
                hi = jnp.where(pred, hi, mid)
                return lo, hi
            lo, hi = lax.fori_loop(0, FINE, fb_body, (lo, hi))
            t_lo = lax.bitcast_convert_type(lo, jnp.float32)

            def s_body(i, acc):
                base = i * FBLK
                for j in range(FBLK // L):
                    v = cb_v[pl.ds(base + j * L, L)]
                    acc = acc + jnp.where(v > t_lo, v, jnp.float32(0.0))
                return acc
            sacc = lax.fori_loop(0, n_blk, s_body, zero)
            return lo, hi, w_hi + jnp.sum(sacc)

        def full_path(carry):
            lo, hi = carry

            def fb_body(_, carry):
                lo, hi = carry
                mid = lo + lax.shift_right_logical(hi - lo, 1)
                t = lax.bitcast_convert_type(mid, jnp.float32)

                def w_body(i, accs):
                    a0, a1 = accs
                    base = i * (UNROLL * L)
                    for j in range(UNROLL):
                        v = row_v[pl.ds(base + j * L, L)]
                        w = jnp.where(v > t, v, jnp.float32(0.0))
                        if j % 2 == 0:
                            a0 = a0 + w
                        else:
                            a1 = a1 + w
                    return a0, a1
                w0, w1 = lax.fori_loop(0, STEPS, w_body, (zero, zero))
                W = jnp.sum(w0 + w1)
                pred = W > t_thresh
                lo = jnp.where(pred, mid, lo)
                hi = jnp.where(pred, hi, mid)
                return lo, hi
            lo, hi = lax.fori_loop(0, FINE, fb_body, (lo, hi))

            def s_body(i, acc):
                base = i * (UNROLL * L)
                t_lo = lax.bitcast_convert_type(lo, jnp.float32)
                for j in range(UNROLL):
                    v = row_v[pl.ds(base + j * L, L)]
                    acc = acc + jnp.where(v > t_lo, v, jnp.float32(0.0))
                return acc
            sacc = lax.fori_loop(0, STEPS, s_body, zero)
            return lo, hi, jnp.sum(sacc)

        lo, hi, kept = lax.cond(cnt <= jnp.int32(CAP), fine_path, full_path,
                                (lo, hi))
        t_lo = lax.bitcast_convert_type(lo, jnp.float32)

        kept_v = jnp.full((L,), 1.0, jnp.float32) * kept
        sp_v = kept_v / z_s
        alpha = jnp.full((L,), 1.0, jnp.float32) / (
            z_s * (sp_v + jnp.float32(1e-7)))

        def out_body(i, _unused):
            base = i * (UNROLL * L)
            for j in range(UNROLL):
                v = row_v[pl.ds(base + j * L, L)]
                row_v[pl.ds(base + j * L, L)] = jnp.where(
                    v > t_lo, v * alpha, jnp.float32(0.0))
            return 0
        lax.fori_loop(0, STEPS, out_body, 0)

    bufs = [b0, b1, b2]
    isems = [si0, si1, si2]
    osems = [so0, so1, so2]
    base_row = wid * ROWS_PER_W
    hin = [None] * ROWS_PER_W
    hout = [None] * ROWS_PER_W
    hin[0] = pltpu.async_copy(x_hbm.at[base_row], bufs[0], isems[0])
    for r in range(ROWS_PER_W):
        if r + 1 < ROWS_PER_W:
            if r + 1 >= 3:
                hout[r - 2].wait()
            hin[r + 1] = pltpu.async_copy(x_hbm.at[base_row + (r + 1)],
                                          bufs[(r + 1) % 3],
                                          isems[(r + 1) % 3])
        hin[r].wait()
        compute_row(bufs[r % 3])
        hout[r] = pltpu.async_copy(bufs[r % 3], out_hbm.at[base_row + r],
                                   osems[r % 3])
    for r in range(max(0, ROWS_PER_W - 3), ROWS_PER_W):
        hout[r].wait()


@jax.jit
def kernel(logits):
    return pl.kernel(
        _body,
        out_type=jax.ShapeDtypeStruct((B, D), jnp.float32),
        mesh=plsc.VectorSubcoreMesh(core_axis_name="c", subcore_axis_name="s"),
        scratch_types=[pltpu.VMEM((D,), jnp.float32),
                       pltpu.VMEM((D,), jnp.float32),
                       pltpu.VMEM((D,), jnp.float32),
                       pltpu.VMEM((CAP + 2 * FBLK,), jnp.float32),
                       pltpu.SemaphoreType.DMA,
                       pltpu.SemaphoreType.DMA,
                       pltpu.SemaphoreType.DMA,
                       pltpu.SemaphoreType.DMA,
                       pltpu.SemaphoreType.DMA,
                       pltpu.SemaphoreType.DMA],
        compiler_params=pltpu.CompilerParams(needs_layout_passes=False),
    )(logits)


# speedup vs baseline: 2.1468x; 1.0984x over previous
import jax
import jax.numpy as jnp
from jax import lax
from jax.experimental import pallas as pl
from jax.experimental.pallas import tpu as pltpu
from jax.experimental.pallas import tpu_sc as plsc

B, D = 128, 32768
L = 16
NC, NS = 2, 16
NW = NC * NS
ROWS_PER_W = B // NW
CHUNKS = D // L
UNROLL = 8
STEPS = CHUNKS // UNROLL
COARSE = 4
FINE = 24
CAP = 8192
FBLK = 128
SHIFT = 20.0


def _body(x_hbm, out_hbm, b0, b1, b2, cb_v, si0, si1, si2, so0, so1, so2):
    c = lax.axis_index("c")
    s = lax.axis_index("s")
    wid = s * NC + c
    zero = jnp.zeros((L,), jnp.float32)
    izero = jnp.zeros((L,), jnp.int32)
    ione = jnp.ones((L,), jnp.int32)

    def compute_row(row_v):
        def exp_body(i, accs):
            a0, a1, q0, q1 = accs
            base = i * (UNROLL * L)
            for j in range(UNROLL):
                xv = row_v[pl.ds(base + j * L, L)]
                u = jnp.exp(jnp.minimum(xv - jnp.float32(SHIFT),
                                        jnp.float32(0.0)))
                row_v[pl.ds(base + j * L, L)] = u
                if j % 2 == 0:
                    a0 = a0 + u
                    q0 = q0 + u * u
                else:
                    a1 = a1 + u
                    q1 = q1 + u * u
            return a0, a1, q0, q1
        z0, z1, q0, q1 = lax.fori_loop(0, STEPS, exp_body,
                                       (zero, zero, zero, zero))
        z_s = jnp.sum(z0 + z1)
        q_s = jnp.sum(q0 + q1)
        t_thresh = jnp.float32(0.5) * z_s

        lo0 = jnp.maximum(
            lax.bitcast_convert_type(z_s * jnp.float32(2.0 ** -16),
                                     jnp.int32) - jnp.int32(16),
            jnp.int32(0))
        ub = ((jnp.full((L,), 2.0, jnp.float32) * q_s) / z_s)[0]
        hi0 = jnp.minimum(lax.bitcast_convert_type(z_s, jnp.int32),
                          lax.bitcast_convert_type(ub, jnp.int32)
                          + jnp.int32(16))

        def bs_body(_, carry):
            lo, hi, above = carry
            mid = lo + lax.shift_right_logical(hi - lo, 1)
            t = lax.bitcast_convert_type(mid, jnp.float32)

            def w_body(i, accs):
                a0, a1, a2, a3 = accs
                base = i * (UNROLL * L)
                for j in range(UNROLL):
                    v = row_v[pl.ds(base + j * L, L)]
                    w = jnp.where(v > t, v, jnp.float32(0.0))
                    if j % 4 == 0:
                        a0 = a0 + w
                    elif j % 4 == 1:
                        a1 = a1 + w
                    elif j % 4 == 2:
                        a2 = a2 + w
                    else:
                        a3 = a3 + w
                return a0, a1, a2, a3
            w0, w1, w2, w3 = lax.fori_loop(0, STEPS, w_body,
                                           (zero, zero, zero, zero))
            W = jnp.sum((w0 + w1) + (w2 + w3))
            pred = W > t_thresh
            lo = jnp.where(pred, mid, lo)
            hi = jnp.where(pred, hi, mid)
            above = jnp.where(pred, above, W)
            return lo, hi, above

        lo, hi, w_hi = lax.fori_loop(0, COARSE, bs_body,
                                     (lo0, hi0, jnp.float32(0.0)))
        lo_f = lax.bitcast_convert_type(lo, jnp.float32)
        hi_f = lax.bitcast_convert_type(hi, jnp.float32)

        def cp_body(i, off):
            base = i * (UNROLL * L)
            vs, ms, pcs = [], [], []
            for j in range(UNROLL):
                v = row_v[pl.ds(base + j * L, L)]
                msk = (v > lo_f) & (v <= hi_f)
                vs.append(v)
                ms.append(msk)
                pcs.append(plsc.all_reduce_population_count(msk))
            pre = [izero]
            for j in range(1, UNROLL):
                pre.append(pre[j - 1] + pcs[j - 1])
            for j in range(UNROLL):
                cs = plsc.cumsum(jnp.where(ms[j], ione, izero))
                pos = (off + pre[j]) + cs
                pos = jnp.minimum(pos - ione, jnp.full((L,), CAP + FBLK - 1,
                                                       jnp.int32))
                plsc.store_scatter(cb_v, [pos], vs[j], mask=ms[j])
            t01 = (pcs[0] + pcs[1]) + (pcs[2] + pcs[3])
            t23 = (pcs[4] + pcs[5]) + (pcs[6] + pcs[7])
            return off + (t01 + t23)
        off = lax.fori_loop(0, STEPS, cp_body, izero)
        cnt = off[0]
        cpos = jnp.minimum(cnt, jnp.int32(CAP))
        for k in range(FBLK // L):
            plsc.store_scatter(
                cb_v, [cpos + (lax.iota(jnp.int32, L) + jnp.int32(k * L))],
                zero)

        def fine_path(carry):
            lo, hi = carry
            n_blk = lax.shift_right_logical(cnt + jnp.int32(FBLK - 1), 7)

            def fb_body(_, carry):
                lo, hi = carry
                mid = lo + lax.shift_right_logical(hi - lo, 1)
                t = lax.bitcast_convert_type(mid, jnp.float32)

                def w_body(i, accs):
                    a0, a1 = accs
                    base = i * FBLK
                    for j in range(FBLK // L):
                        v = cb_v[pl.ds(base + j * L, L)]
                        w = jnp.where(v > t, v, jnp.float32(0.0))
                        if j % 2 == 0:
                            a0 = a0 + w
                        else:
                            a1 = a1 + w
                    return a0, a1
                w0, w1 = lax.fori_loop(0, n_blk, w_body, (zero, zero))
                W = w_hi + jnp.sum(w0 + w1)
                pred = W > t_thresh
                lo = jnp.where(pred, mid, lo)
                hi = jnp.where(pred, hi, mid)
                return lo, hi
            lo, hi = lax.fori_loop(0, FINE, fb_body, (lo, hi))
            t_lo = lax.bitcast_convert_type(lo, jnp.float32)

            def s_body(i, acc):
                base = i * FBLK
                for j in range(FBLK // L):
                    v = cb_v[pl.ds(base + j * L, L)]
                    acc = acc + jnp.where(v > t_lo, v, jnp.float32(0.0))
                return acc
            sacc = lax.fori_loop(0, n_blk, s_body, zero)
            return lo, hi, w_hi + jnp.sum(sacc)

        def full_path(carry):
            lo, hi = carry

            def fb_body(_, carry):
                lo, hi = carry
                mid = lo + lax.shift_right_logical(hi - lo, 1)
                t = lax.bitcast_convert_type(mid, jnp.float32)

                def w_body(i, accs):
                    a0, a1 = accs
                    base = i * (UNROLL * L)
                    for j in range(UNROLL):
                        v = row_v[pl.ds(base + j * L, L)]
                        w = jnp.where(v > t, v, jnp.float32(0.0))
                        if j % 2 == 0:
                            a0 = a0 + w
                        else:
                            a1 = a1 + w
                    return a0, a1
                w0, w1 = lax.fori_loop(0, STEPS, w_body, (zero, zero))
                W = jnp.sum(w0 + w1)
                pred = W > t_thresh
                lo = jnp.where(pred, mid, lo)
                hi = jnp.where(pred, hi, mid)
                return lo, hi
            lo, hi = lax.fori_loop(0, FINE, fb_body, (lo, hi))

            def s_body(i, acc):
                base = i * (UNROLL * L)
                t_lo = lax.bitcast_convert_type(lo, jnp.float32)
                for j in range(UNROLL):
                    v = row_v[pl.ds(base + j * L, L)]
                    acc = acc + jnp.where(v > t_lo, v, jnp.float32(0.0))
                return acc
            sacc = lax.fori_loop(0, STEPS, s_body, zero)
            return lo, hi, jnp.sum(sacc)

        lo, hi, kept = lax.cond(cnt <= jnp.int32(CAP), fine_path, full_path,
                                (lo, hi))
        t_lo = lax.bitcast_convert_type(lo, jnp.float32)

        kept_v = jnp.full((L,), 1.0, jnp.float32) * kept
        sp_v = kept_v / z_s
        alpha = jnp.full((L,), 1.0, jnp.float32) / (
            z_s * (sp_v + jnp.float32(1e-7)))

        def out_body(i, _unused):
            base = i * (UNROLL * L)
            for j in range(UNROLL):
                v = row_v[pl.ds(base + j * L, L)]
                row_v[pl.ds(base + j * L, L)] = jnp.where(
                    v > t_lo, v * alpha, jnp.float32(0.0))
            return 0
        lax.fori_loop(0, STEPS, out_body, 0)

    bufs = [b0, b1, b2]
    isems = [si0, si1, si2]
    osems = [so0, so1, so2]
    base_row = wid * ROWS_PER_W
    hin = [None] * ROWS_PER_W
    hout = [None] * ROWS_PER_W
    hin[0] = pltpu.async_copy(x_hbm.at[base_row], bufs[0], isems[0])
    for r in range(ROWS_PER_W):
        if r + 1 < ROWS_PER_W:
            if r + 1 >= 3:
                hout[r - 2].wait()
            hin[r + 1] = pltpu.async_copy(x_hbm.at[base_row + (r + 1)],
                                          bufs[(r + 1) % 3],
                                          isems[(r + 1) % 3])
        hin[r].wait()
        compute_row(bufs[r % 3])
        hout[r] = pltpu.async_copy(bufs[r % 3], out_hbm.at[base_row + r],
                                   osems[r % 3])
    for r in range(max(0, ROWS_PER_W - 3), ROWS_PER_W):
        hout[r].wait()


@jax.jit
def kernel(logits):
    return pl.kernel(
        _body,
        out_type=jax.ShapeDtypeStruct((B, D), jnp.float32),
        mesh=plsc.VectorSubcoreMesh(core_axis_name="c", subcore_axis_name="s"),
        scratch_types=[pltpu.VMEM((D,), jnp.float32),
                       pltpu.VMEM((D,), jnp.float32),
                       pltpu.VMEM((D,), jnp.float32),
                       pltpu.VMEM((CAP + 2 * FBLK,), jnp.float32),
                       pltpu.SemaphoreType.DMA,
                       pltpu.SemaphoreType.DMA,
                       pltpu.SemaphoreType.DMA,
                       pltpu.SemaphoreType.DMA,
                       pltpu.SemaphoreType.DMA,
                       pltpu.SemaphoreType.DMA],
        compiler_params=pltpu.CompilerParams(needs_layout_passes=False),
    )(logits)
